# Initial kernel scaffold; baseline (speedup 1.0000x reference)
import jax
import jax.numpy as jnp
from jax.experimental import pallas as pl


def kernel(x, edge_index, edge_attr, W1, b1, W2, b2, W_root, b_conv, W_ih, W_hh, b_ih, b_hh):
    raise NotImplementedError("write your pallas kernel here")


# speedup vs baseline: 3.0230x; 3.0230x over previous
import functools

import jax
import jax.numpy as jnp
from jax import lax
from jax.experimental import pallas as pl
from jax.experimental.pallas import tpu as pltpu
from jax.experimental.pallas import tpu_sc as plsc

f32 = jnp.float32
i32 = jnp.int32

NC = 2
NS = 16
NW = NC * NS
C = 128


def _gather_kernel(N, D, NCH):
    mesh = plsc.VectorSubcoreMesh(core_axis_name="c", subcore_axis_name="s",
                                  num_cores=NC, num_subcores=NS)

    @functools.partial(
        pl.kernel,
        out_type=jax.ShapeDtypeStruct((NW * NCH, C, D), f32),
        mesh=mesh,
        compiler_params=pltpu.CompilerParams(use_tc_tiling_on_sc=False),
        scratch_types=[
            pltpu.VMEM((NCH, C), i32),
            pltpu.VMEM((C, D), f32),
            pltpu.SemaphoreType.DMA,
        ],
    )
    def gk(x_hbm, src_hbm, out_hbm, idx_v, rows_v, sem):
        wid = lax.axis_index("s") * NC + lax.axis_index("c")
        pltpu.sync_copy(src_hbm.at[wid], idx_v)

        def body(j, carry):
            pltpu.async_copy(x_hbm.at[idx_v.at[j]], rows_v, sem).wait()
            pltpu.sync_copy(rows_v, out_hbm.at[wid * NCH + j])
            return carry

        lax.fori_loop(0, NCH, body, 0)

    return gk


def _scatter_kernel(NP, D, NCH, RPS):
    mesh = plsc.VectorSubcoreMesh(core_axis_name="c", subcore_axis_name="s",
                                  num_cores=NC, num_subcores=NS)

    @functools.partial(
        pl.kernel,
        out_type=(
            jax.ShapeDtypeStruct((NC * NS, RPS, D), f32),
            jax.ShapeDtypeStruct((NC * NS, RPS, 16), f32),
        ),
        mesh=mesh,
        compiler_params=pltpu.CompilerParams(use_tc_tiling_on_sc=False),
        scratch_types=[
            pltpu.VMEM((NCH, C), i32),
            pltpu.VMEM((C, D), f32),
            pltpu.VMEM((C, 16), f32),
            pltpu.VMEM((RPS, D), f32),
            pltpu.VMEM((RPS, 16), f32),
            pltpu.VMEM_SHARED((NP, D), f32),
            pltpu.VMEM_SHARED((NP, 16), f32),
        ],
    )
    def sk(msg_hbm, dst_hbm, agg_hbm, cnt_hbm,
           idx_v, rows_v, ones_v, zb, zc, agg_s, cnt_s):
        cid = lax.axis_index("c")
        sid = lax.axis_index("s")
        wid = sid * NC + cid

        z16 = jnp.zeros((16,), f32)
        o16 = jnp.ones((16,), f32)

        def fz(i, carry):
            for t in range(D // 16):
                zb[i, pl.ds(16 * t, 16)] = z16
            zc[i, :] = z16
            return carry

        lax.fori_loop(0, RPS, fz, 0)

        def fo(i, carry):
            ones_v[i, :] = o16
            return carry

        lax.fori_loop(0, C, fo, 0)

        pltpu.sync_copy(zb, agg_s.at[pl.ds(sid * RPS, RPS)])
        pltpu.sync_copy(zc, cnt_s.at[pl.ds(sid * RPS, RPS)])
        pltpu.sync_copy(dst_hbm.at[wid], idx_v)
        plsc.subcore_barrier()

        def body(j, carry):
            pltpu.sync_copy(msg_hbm.at[wid * NCH + j], rows_v)
            pltpu.sync_copy(rows_v, agg_s.at[idx_v.at[j]], add=True)
            pltpu.sync_copy(ones_v, cnt_s.at[idx_v.at[j]], add=True)
            return carry

        lax.fori_loop(0, NCH, body, 0)
        plsc.subcore_barrier()

        out_row = cid * NS + sid
        pltpu.sync_copy(agg_s.at[pl.ds(sid * RPS, RPS)], agg_hbm.at[out_row])
        pltpu.sync_copy(cnt_s.at[pl.ds(sid * RPS, RPS)], cnt_hbm.at[out_row])

    return sk


def _msg_body(ED, D, HNN, ea_ref, xj_ref, w1t_ref, b1_ref, w2pt_ref, b2t_ref,
              out_ref):
    he = jnp.dot(w1t_ref[...], ea_ref[...], preferred_element_type=f32)
    he = jnp.maximum(he + b1_ref[...], 0.0)
    xj = xj_ref[...]
    q = jnp.dot(w2pt_ref[...], xj, preferred_element_type=f32)
    acc = jnp.dot(b2t_ref[...], xj, preferred_element_type=f32)
    for h in range(HNN):
        acc = acc + q[h * D:(h + 1) * D, :] * he[h:h + 1, :]
    out_ref[...] = acc


def _tail_body(D, x_ref, agg_ref, cnt_ref, wr_ref, bc_ref, wih_ref, bih_ref,
               whh_ref, bhh_ref, xo_ref, hn_ref):
    xb = x_ref[...]
    s = agg_ref[0] + agg_ref[1]
    c = cnt_ref[0, :, 0:1] + cnt_ref[1, :, 0:1]
    mean = s / jnp.maximum(c, 1.0)
    conv = mean + jnp.dot(xb, wr_ref[...], preferred_element_type=f32)
    conv = conv + bc_ref[...]
    xin = jnp.maximum(conv, 0.0) + (jnp.exp(jnp.minimum(conv, 0.0)) - 1.0)
    gi = jnp.dot(xin, wih_ref[...], preferred_element_type=f32) + bih_ref[...]
    gh = jnp.dot(xb, whh_ref[...], preferred_element_type=f32) + bhh_ref[...]
    r = jax.nn.sigmoid(gi[:, :D] + gh[:, :D])
    z = jax.nn.sigmoid(gi[:, D:2 * D] + gh[:, D:2 * D])
    n = jnp.tanh(gi[:, 2 * D:] + r * gh[:, 2 * D:])
    hn = (1.0 - z) * n + z * xb
    hn_ref[...] = hn
    xo_ref[...] = jnp.maximum(hn + xb, 0.0)


def kernel(x, edge_index, edge_attr, W1, b1, W2, b2, W_root, b_conv,
           W_ih, W_hh, b_ih, b_hh):
    N, D = x.shape
    E = edge_index.shape[1]
    ED = edge_attr.shape[1]
    HNN = W1.shape[1]

    NCH = -(-E // (NW * C))
    EP = NW * NCH * C
    RPS = -(-(N + 1) // NS)
    RPS = ((RPS + 7) // 8) * 8
    NP = NS * RPS
    pad = EP - E

    src = edge_index[0]
    dst = edge_index[1]
    src_p = jnp.concatenate([src, jnp.zeros((pad,), i32)]).reshape(NW, NCH, C)
    dst_p = jnp.concatenate([dst, jnp.full((pad,), N, i32)]).reshape(NW, NCH, C)

    xg = _gather_kernel(N, D, NCH)(x, src_p)
    xjT = xg.reshape(EP, D)[:E].T

    eaT = edge_attr.T
    W1T = W1.T
    b1c = b1[:, None]
    W2pT = W2.reshape(HNN, D, D).transpose(0, 2, 1).reshape(HNN * D, D)
    B2T = b2.reshape(D, D).T

    BE = 1280
    n_blk = E // BE
    msgT = pl.pallas_call(
        functools.partial(_msg_body, ED, D, HNN),
        grid=(n_blk,),
        in_specs=[
            pl.BlockSpec((ED, BE), lambda i: (0, i)),
            pl.BlockSpec((D, BE), lambda i: (0, i)),
            pl.BlockSpec((HNN, ED), lambda i: (0, 0)),
            pl.BlockSpec((HNN, 1), lambda i: (0, 0)),
            pl.BlockSpec((HNN * D, D), lambda i: (0, 0)),
            pl.BlockSpec((D, D), lambda i: (0, 0)),
        ],
        out_specs=pl.BlockSpec((D, BE), lambda i: (0, i)),
        out_shape=jax.ShapeDtypeStruct((D, E), f32),
    )(eaT, xjT, W1T, b1c, W2pT, B2T)

    msg_p = jnp.concatenate([msgT.T, jnp.zeros((pad, D), f32)])
    msg_p = msg_p.reshape(NW * NCH, C, D)

    agg_out, cnt_out = _scatter_kernel(NP, D, NCH, RPS)(msg_p, dst_p)
    agg2 = agg_out.reshape(NC, NP, D)[:, :N]
    cnt2 = cnt_out.reshape(NC, NP, 16)[:, :N]

    BN = 2000
    n_blk2 = N // BN
    x_out, h_new = pl.pallas_call(
        functools.partial(_tail_body, D),
        grid=(n_blk2,),
        in_specs=[
            pl.BlockSpec((BN, D), lambda i: (i, 0)),
            pl.BlockSpec((NC, BN, D), lambda i: (0, i, 0)),
            pl.BlockSpec((NC, BN, 16), lambda i: (0, i, 0)),
            pl.BlockSpec((D, D), lambda i: (0, 0)),
            pl.BlockSpec((1, D), lambda i: (0, 0)),
            pl.BlockSpec((D, 3 * D), lambda i: (0, 0)),
            pl.BlockSpec((1, 3 * D), lambda i: (0, 0)),
            pl.BlockSpec((D, 3 * D), lambda i: (0, 0)),
            pl.BlockSpec((1, 3 * D), lambda i: (0, 0)),
        ],
        out_specs=[
            pl.BlockSpec((BN, D), lambda i: (i, 0)),
            pl.BlockSpec((BN, D), lambda i: (i, 0)),
        ],
        out_shape=[
            jax.ShapeDtypeStruct((N, D), f32),
            jax.ShapeDtypeStruct((N, D), f32),
        ],
    )(x, agg2, cnt2, W_root, b_conv[None, :], W_ih.T, b_ih[None, :],
      W_hh.T, b_hh[None, :])

    return (x_out, h_new[None, :, :])


# speedup vs baseline: 3.9691x; 1.3130x over previous
import functools

import jax
import jax.numpy as jnp
from jax import lax
from jax.experimental import pallas as pl
from jax.experimental.pallas import tpu as pltpu
from jax.experimental.pallas import tpu_sc as plsc

f32 = jnp.float32
i32 = jnp.int32

NC = 2
NS = 16
NW = NC * NS
C = 128


def _gather_kernel(N, D, NCH):
    mesh = plsc.VectorSubcoreMesh(core_axis_name="c", subcore_axis_name="s",
                                  num_cores=NC, num_subcores=NS)

    @functools.partial(
        pl.kernel,
        out_type=jax.ShapeDtypeStruct((NW * NCH, C, D), f32),
        mesh=mesh,
        compiler_params=pltpu.CompilerParams(use_tc_tiling_on_sc=False),
        scratch_types=[
            pltpu.VMEM((NCH, C), i32),
            pltpu.VMEM((C, D), f32),
            pltpu.VMEM((C, D), f32),
            pltpu.VMEM((C, D), f32),
            pltpu.VMEM((C, D), f32),
            pltpu.SemaphoreType.DMA,
            pltpu.SemaphoreType.DMA,
            pltpu.SemaphoreType.DMA,
            pltpu.SemaphoreType.DMA,
        ],
    )
    def gk(x_hbm, src_hbm, out_hbm, idx_v, r0, r1, r2, r3, g0, g1, g2, g3):
        wid = lax.axis_index("s") * NC + lax.axis_index("c")
        pltpu.sync_copy(src_hbm.at[wid], idx_v)
        rows = (r0, r1, r2, r3)
        sems = (g0, g1, g2, g3)

        def body(t, carry):
            j = 4 * t
            ds = [pltpu.async_copy(x_hbm.at[idx_v.at[j + i]], rows[i], sems[i])
                  for i in range(4)]
            for i in range(4):
                ds[i].wait()
                pltpu.sync_copy(rows[i], out_hbm.at[wid * NCH + j + i])
            return carry

        lax.fori_loop(0, NCH // 4, body, 0)

    return gk


def _scatter_kernel(NP, D, NCH, RPS):
    mesh = plsc.VectorSubcoreMesh(core_axis_name="c", subcore_axis_name="s",
                                  num_cores=NC, num_subcores=NS)

    @functools.partial(
        pl.kernel,
        out_type=(
            jax.ShapeDtypeStruct((NC * NS, RPS, D), f32),
            jax.ShapeDtypeStruct((NC * NS, RPS, 16), f32),
        ),
        mesh=mesh,
        compiler_params=pltpu.CompilerParams(use_tc_tiling_on_sc=False),
        scratch_types=[
            pltpu.VMEM((NCH, C), i32),
            pltpu.VMEM((C, D), f32),
            pltpu.VMEM((C, D), f32),
            pltpu.VMEM((C, D), f32),
            pltpu.VMEM((C, D), f32),
            pltpu.VMEM((C, 16), f32),
            pltpu.VMEM((RPS, D), f32),
            pltpu.VMEM((RPS, 16), f32),
            pltpu.VMEM_SHARED((NP, D), f32),
            pltpu.VMEM_SHARED((NP, 16), f32),
            pltpu.SemaphoreType.DMA,
            pltpu.SemaphoreType.DMA,
            pltpu.SemaphoreType.DMA,
            pltpu.SemaphoreType.DMA,
            pltpu.SemaphoreType.DMA,
            pltpu.SemaphoreType.DMA,
            pltpu.SemaphoreType.DMA,
            pltpu.SemaphoreType.DMA,
        ],
    )
    def sk(msg_hbm, dst_hbm, agg_hbm, cnt_hbm,
           idx_v, r0, r1, r2, r3, ones_v, zb, zc, agg_s, cnt_s,
           l0, l1, l2, l3, a0, a1, a2, a3):
        cid = lax.axis_index("c")
        sid = lax.axis_index("s")
        wid = sid * NC + cid

        z16 = jnp.zeros((16,), f32)
        o16 = jnp.ones((16,), f32)

        def fz(i, carry):
            for t in range(D // 16):
                zb[i, pl.ds(16 * t, 16)] = z16
            zc[i, :] = z16
            return carry

        lax.fori_loop(0, RPS, fz, 0)

        def fo(i, carry):
            ones_v[i, :] = o16
            return carry

        lax.fori_loop(0, C, fo, 0)

        pltpu.sync_copy(zb, agg_s.at[pl.ds(sid * RPS, RPS)])
        pltpu.sync_copy(zc, cnt_s.at[pl.ds(sid * RPS, RPS)])
        pltpu.sync_copy(dst_hbm.at[wid], idx_v)
        plsc.subcore_barrier()

        rows = (r0, r1, r2, r3)
        lsem = (l0, l1, l2, l3)
        asem = (a0, a1, a2, a3)

        def body(t, carry):
            j = 4 * t
            lds = [pltpu.async_copy(msg_hbm.at[wid * NCH + j + i], rows[i],
                                    lsem[i]) for i in range(4)]
            sca = []
            for i in range(4):
                lds[i].wait()
                sca.append(pltpu.async_copy(
                    rows[i], agg_s.at[idx_v.at[j + i]], asem[i], add=True))
                pltpu.sync_copy(ones_v, cnt_s.at[idx_v.at[j + i]], add=True)
            for i in range(4):
                sca[i].wait()
            return carry

        lax.fori_loop(0, NCH // 4, body, 0)
        plsc.subcore_barrier()

        out_row = cid * NS + sid
        pltpu.sync_copy(agg_s.at[pl.ds(sid * RPS, RPS)], agg_hbm.at[out_row])
        pltpu.sync_copy(cnt_s.at[pl.ds(sid * RPS, RPS)], cnt_hbm.at[out_row])

    return sk


def _msg_body(ED, D, HNN, ea_ref, xj_ref, w1t_ref, b1_ref, w2ct_ref,
              out_ref, u_ref):
    eaT = ea_ref[...].T
    he = jnp.dot(w1t_ref[...], eaT, preferred_element_type=f32)
    he = jnp.maximum(he + b1_ref[...], 0.0)
    xjT = xj_ref[...].T
    for h in range(HNN):
        u_ref[h * D:(h + 1) * D, :] = xjT * he[h:h + 1, :]
    u_ref[HNN * D:HNN * D + D, :] = xjT
    msgT = jnp.dot(w2ct_ref[...], u_ref[...], preferred_element_type=f32)
    out_ref[...] = msgT.T


def _tail_body(D, x_ref, agg_ref, cnt_ref, wr_ref, bc_ref, wih_ref, bih_ref,
               whh_ref, bhh_ref, xo_ref, hn_ref):
    xb = x_ref[...]
    s = agg_ref[0] + agg_ref[1]
    c = cnt_ref[0, :, 0:1] + cnt_ref[1, :, 0:1]
    mean = s / jnp.maximum(c, 1.0)
    conv = mean + jnp.dot(xb, wr_ref[...], preferred_element_type=f32)
    conv = conv + bc_ref[...]
    xin = jnp.maximum(conv, 0.0) + (jnp.exp(jnp.minimum(conv, 0.0)) - 1.0)
    gi = jnp.dot(xin, wih_ref[...], preferred_element_type=f32) + bih_ref[...]
    gh = jnp.dot(xb, whh_ref[...], preferred_element_type=f32) + bhh_ref[...]
    r = jax.nn.sigmoid(gi[:, :D] + gh[:, :D])
    z = jax.nn.sigmoid(gi[:, D:2 * D] + gh[:, D:2 * D])
    n = jnp.tanh(gi[:, 2 * D:] + r * gh[:, 2 * D:])
    hn = (1.0 - z) * n + z * xb
    hn_ref[...] = hn
    xo_ref[...] = jnp.maximum(hn + xb, 0.0)


def kernel(x, edge_index, edge_attr, W1, b1, W2, b2, W_root, b_conv,
           W_ih, W_hh, b_ih, b_hh):
    N, D = x.shape
    E = edge_index.shape[1]
    ED = edge_attr.shape[1]
    HNN = W1.shape[1]

    NCH = -(-E // (NW * C))
    EP = NW * NCH * C
    RPS = -(-(N + 1) // NS)
    RPS = ((RPS + 7) // 8) * 8
    NP = NS * RPS
    pad = EP - E

    src = edge_index[0]
    dst = edge_index[1]
    src_p = jnp.concatenate([src, jnp.zeros((pad,), i32)]).reshape(NW, NCH, C)
    dst_p = jnp.concatenate([dst, jnp.full((pad,), N, i32)]).reshape(NW, NCH, C)

    xg = _gather_kernel(N, D, NCH)(x, src_p)
    x_j = xg.reshape(EP, D)

    W1T = W1.T
    b1c = b1[:, None]
    W2c = W2.reshape(HNN, D, D).transpose(2, 0, 1).reshape(D, HNN * D)
    W2full = jnp.concatenate([W2c, b2.reshape(D, D).T], axis=1)

    BE = 1280
    n_blk = E // BE
    msg_p = pl.pallas_call(
        functools.partial(_msg_body, ED, D, HNN),
        grid=(n_blk,),
        in_specs=[
            pl.BlockSpec((BE, ED), lambda i: (i, 0)),
            pl.BlockSpec((BE, D), lambda i: (i, 0)),
            pl.BlockSpec((HNN, ED), lambda i: (0, 0)),
            pl.BlockSpec((HNN, 1), lambda i: (0, 0)),
            pl.BlockSpec((D, HNN * D + D), lambda i: (0, 0)),
        ],
        out_specs=pl.BlockSpec((BE, D), lambda i: (i, 0)),
        out_shape=jax.ShapeDtypeStruct((EP, D), f32),
        scratch_shapes=[pltpu.VMEM((HNN * D + D, BE), f32)],
    )(edge_attr, x_j, W1T, b1c, W2full)

    msg_p = msg_p.reshape(NW * NCH, C, D)

    agg_out, cnt_out = _scatter_kernel(NP, D, NCH, RPS)(msg_p, dst_p)
    agg2 = agg_out.reshape(NC, NP, D)[:, :N]
    cnt2 = cnt_out.reshape(NC, NP, 16)[:, :N]

    BN = 2000
    n_blk2 = N // BN
    x_out, h_new = pl.pallas_call(
        functools.partial(_tail_body, D),
        grid=(n_blk2,),
        in_specs=[
            pl.BlockSpec((BN, D), lambda i: (i, 0)),
            pl.BlockSpec((NC, BN, D), lambda i: (0, i, 0)),
            pl.BlockSpec((NC, BN, 16), lambda i: (0, i, 0)),
            pl.BlockSpec((D, D), lambda i: (0, 0)),
            pl.BlockSpec((1, D), lambda i: (0, 0)),
            pl.BlockSpec((D, 3 * D), lambda i: (0, 0)),
            pl.BlockSpec((1, 3 * D), lambda i: (0, 0)),
            pl.BlockSpec((D, 3 * D), lambda i: (0, 0)),
            pl.BlockSpec((1, 3 * D), lambda i: (0, 0)),
        ],
        out_specs=[
            pl.BlockSpec((BN, D), lambda i: (i, 0)),
            pl.BlockSpec((BN, D), lambda i: (i, 0)),
        ],
        out_shape=[
            jax.ShapeDtypeStruct((N, D), f32),
            jax.ShapeDtypeStruct((N, D), f32),
        ],
    )(x, agg2, cnt2, W_root, b_conv[None, :], W_ih.T, b_ih[None, :],
      W_hh.T, b_hh[None, :])

    return (x_out, h_new[None, :, :])
